# baseline (speedup 1.0000x reference)
import jax
import jax.numpy as jnp
from jax.experimental import pallas as pl
from jax.experimental.pallas import tpu as pltpu

N = 32768
D = 768
BLK = 1024
NBLK = N // BLK


def _router_body(u_ref, w1_ref, b1_ref, w2_ref, b2_ref, w3_ref, b3_ref,
                 mask_ref, flags_ref):
    u = u_ref[...]
    umin = jnp.min(u)
    umax = jnp.max(u)
    un = (u - umin) / (umax - umin + 1e-8)
    acc = [b2_ref[0, k] * jnp.ones_like(un) for k in range(16)]
    for j in range(32):
        hj = jax.nn.relu(un * w1_ref[0, j] + b1_ref[0, j])
        for k in range(16):
            acc[k] = acc[k] + hj * w2_ref[j, k]
    l0 = jnp.full_like(un, b3_ref[0, 0])
    l1 = jnp.full_like(un, b3_ref[0, 1])
    l2 = jnp.full_like(un, b3_ref[0, 2])
    for k in range(16):
        hk = jax.nn.relu(acc[k])
        l0 = l0 + hk * w3_ref[k, 0]
        l1 = l1 + hk * w3_ref[k, 1]
        l2 = l2 + hk * w3_ref[k, 2]
    d = jnp.where((l1 > l0) & (l1 >= l2), 1.0,
                  jnp.where((l2 > l0) & (l2 > l1), 2.0, 0.0))
    mask_ref[...] = d
    any0 = jnp.max(jnp.where(d == 0.0, 1.0, 0.0), axis=1, keepdims=True)
    any1 = jnp.max(jnp.where(d == 1.0, 1.0, 0.0), axis=1, keepdims=True)
    any2 = jnp.max(jnp.where(d == 2.0, 1.0, 0.0), axis=1, keepdims=True)
    flags_ref[...] = any0 + 2.0 * any1 + 4.0 * any2


def _fuse_body(w0_ref, b0_ref, w1_ref, b1_ref, w2_ref, b2_ref, w3_ref, b3_ref,
               wc1_ref, bc1_ref, wc2_ref, bc2_ref):
    w01 = jnp.dot(w0_ref[...], w1_ref[...], preferred_element_type=jnp.float32)
    wc1_ref[...] = w01
    bc1 = jnp.dot(b0_ref[...], w1_ref[...], preferred_element_type=jnp.float32) + b1_ref[...]
    bc1_ref[...] = bc1
    w012 = jnp.dot(w01, w2_ref[...], preferred_element_type=jnp.float32)
    wc2_ref[...] = jnp.dot(w012, w3_ref[...], preferred_element_type=jnp.float32)
    bc2 = jnp.dot(bc1, w2_ref[...], preferred_element_type=jnp.float32) + b2_ref[...]
    bc2_ref[...] = jnp.dot(bc2, w3_ref[...], preferred_element_type=jnp.float32) + b3_ref[...]


def _chain_body(flags_ref, x_ref, m_ref,
                w0_ref, b0_ref, wc1_ref, bc1_ref, wc2_ref, bc2_ref,
                out_ref):
    i = pl.program_id(0)
    f = flags_ref[i]
    x = x_ref[...]
    m = m_ref[...]
    zeros = jnp.zeros((BLK, D), dtype=jnp.float32)

    s0 = jax.lax.cond(
        (f & 1) != 0,
        lambda: jnp.dot(x, w0_ref[...], preferred_element_type=jnp.float32) + b0_ref[...],
        lambda: zeros)
    s1 = jax.lax.cond(
        (f & 2) != 0,
        lambda: jnp.dot(x, wc1_ref[...], preferred_element_type=jnp.float32) + bc1_ref[...],
        lambda: zeros)
    s2 = jax.lax.cond(
        (f & 4) != 0,
        lambda: jnp.dot(x, wc2_ref[...], preferred_element_type=jnp.float32) + bc2_ref[...],
        lambda: zeros)
    out_ref[...] = jnp.where(m == 0.0, s0, jnp.where(m == 1.0, s1, s2))


_INTERPRET = False


def _full(shape):
    return pl.BlockSpec(shape, lambda i, flags: (0, 0))


def kernel(x, current_uncertainty, rW1, rb1, rW2, rb2, rW3, rb3,
           W0, b0, W1, b1, W2, b2, W3, b3):
    u2 = current_uncertainty.reshape(NBLK, BLK)
    mask2, flags2 = pl.pallas_call(
        _router_body,
        out_shape=(jax.ShapeDtypeStruct((NBLK, BLK), jnp.float32),
                   jax.ShapeDtypeStruct((NBLK, 1), jnp.float32)),
        interpret=_INTERPRET,
    )(u2, rW1, rb1.reshape(1, 32), rW2, rb2.reshape(1, 16), rW3,
      rb3.reshape(1, 3))

    flags = flags2.reshape(NBLK).astype(jnp.int32)
    mask = mask2.reshape(N)

    wc1, bc1, wc2, bc2 = pl.pallas_call(
        _fuse_body,
        out_shape=(jax.ShapeDtypeStruct((D, D), jnp.float32),
                   jax.ShapeDtypeStruct((1, D), jnp.float32),
                   jax.ShapeDtypeStruct((D, D), jnp.float32),
                   jax.ShapeDtypeStruct((1, D), jnp.float32)),
        interpret=_INTERPRET,
    )(W0, b0.reshape(1, D), W1, b1.reshape(1, D),
      W2, b2.reshape(1, D), W3, b3.reshape(1, D))

    grid_spec = pltpu.PrefetchScalarGridSpec(
        num_scalar_prefetch=1,
        grid=(NBLK,),
        in_specs=[
            pl.BlockSpec((BLK, D), lambda i, flags: (i, 0)),
            pl.BlockSpec((BLK, 1), lambda i, flags: (i, 0)),
            _full((D, D)), _full((1, D)),
            _full((D, D)), _full((1, D)),
            _full((D, D)), _full((1, D)),
        ],
        out_specs=pl.BlockSpec((BLK, D), lambda i, flags: (i, 0)),
    )
    out = pl.pallas_call(
        _chain_body,
        grid_spec=grid_spec,
        out_shape=jax.ShapeDtypeStruct((N, D), jnp.float32),
        interpret=_INTERPRET,
    )(flags, x, mask.reshape(N, 1),
      W0, b0.reshape(1, D), wc1, bc1, wc2, bc2)
    return out, mask


# speedup vs baseline: 1.6500x; 1.6500x over previous
import jax
import jax.numpy as jnp
from jax.experimental import pallas as pl
from jax.experimental.pallas import tpu as pltpu

N = 32768
D = 768
BLK = 1024
NBLK = N // BLK


def _router_body(u_ref, w1_ref, b1_ref, w2_ref, b2_ref, w3_ref, b3_ref,
                 mask_ref, flags_ref):
    u = u_ref[...]
    umin = jnp.min(u)
    umax = jnp.max(u)
    un = (u - umin) / (umax - umin + 1e-8)
    acc = [b2_ref[0, k] * jnp.ones_like(un) for k in range(16)]
    for j in range(32):
        hj = jax.nn.relu(un * w1_ref[0, j] + b1_ref[0, j])
        for k in range(16):
            acc[k] = acc[k] + hj * w2_ref[j, k]
    l0 = jnp.full_like(un, b3_ref[0, 0])
    l1 = jnp.full_like(un, b3_ref[0, 1])
    l2 = jnp.full_like(un, b3_ref[0, 2])
    for k in range(16):
        hk = jax.nn.relu(acc[k])
        l0 = l0 + hk * w3_ref[k, 0]
        l1 = l1 + hk * w3_ref[k, 1]
        l2 = l2 + hk * w3_ref[k, 2]
    d = jnp.where((l1 > l0) & (l1 >= l2), 1.0,
                  jnp.where((l2 > l0) & (l2 > l1), 2.0, 0.0))
    mask_ref[...] = d
    any0 = jnp.max(jnp.where(d == 0.0, 1.0, 0.0), axis=1, keepdims=True)
    any1 = jnp.max(jnp.where(d == 1.0, 1.0, 0.0), axis=1, keepdims=True)
    any2 = jnp.max(jnp.where(d == 2.0, 1.0, 0.0), axis=1, keepdims=True)
    flags_ref[...] = any0 + 2.0 * any1 + 4.0 * any2


def _fuse_body(w0_ref, b0_ref, w1_ref, b1_ref, w2_ref, b2_ref, w3_ref, b3_ref,
               wc1_ref, bc1_ref, wc2_ref, bc2_ref):
    w01 = jnp.dot(w0_ref[...], w1_ref[...], preferred_element_type=jnp.float32)
    wc1_ref[...] = w01
    bc1 = jnp.dot(b0_ref[...], w1_ref[...], preferred_element_type=jnp.float32) + b1_ref[...]
    bc1_ref[...] = bc1
    w012 = jnp.dot(w01, w2_ref[...], preferred_element_type=jnp.float32)
    wc2_ref[...] = jnp.dot(w012, w3_ref[...], preferred_element_type=jnp.float32)
    bc2 = jnp.dot(bc1, w2_ref[...], preferred_element_type=jnp.float32) + b2_ref[...]
    bc2_ref[...] = jnp.dot(bc2, w3_ref[...], preferred_element_type=jnp.float32) + b3_ref[...]


def _chain_body(flags_ref, x_ref, m_ref,
                w0_ref, b0_ref, wc1_ref, bc1_ref, wc2_ref, bc2_ref,
                out_ref):
    i = pl.program_id(0)
    f = flags_ref[i]
    x = x_ref[...]
    m = m_ref[...]

    @pl.when((f & 1) != 0)
    def _():
        out_ref[...] = jnp.dot(x, w0_ref[...], preferred_element_type=jnp.float32) + b0_ref[...]

    @pl.when((f & 2) != 0)
    def _():
        s1 = jnp.dot(x, wc1_ref[...], preferred_element_type=jnp.float32) + bc1_ref[...]

        @pl.when((f & 1) != 0)
        def _():
            out_ref[...] = jnp.where(m == 1.0, s1, out_ref[...])

        @pl.when((f & 1) == 0)
        def _():
            out_ref[...] = s1

    @pl.when((f & 4) != 0)
    def _():
        s2 = jnp.dot(x, wc2_ref[...], preferred_element_type=jnp.float32) + bc2_ref[...]

        @pl.when((f & 3) != 0)
        def _():
            out_ref[...] = jnp.where(m == 2.0, s2, out_ref[...])

        @pl.when((f & 3) == 0)
        def _():
            out_ref[...] = s2


_INTERPRET = False


def _full(shape):
    return pl.BlockSpec(shape, lambda i, flags: (0, 0))


def kernel(x, current_uncertainty, rW1, rb1, rW2, rb2, rW3, rb3,
           W0, b0, W1, b1, W2, b2, W3, b3):
    u2 = current_uncertainty.reshape(NBLK, BLK)
    mask2, flags2 = pl.pallas_call(
        _router_body,
        out_shape=(jax.ShapeDtypeStruct((NBLK, BLK), jnp.float32),
                   jax.ShapeDtypeStruct((NBLK, 1), jnp.float32)),
        interpret=_INTERPRET,
    )(u2, rW1, rb1.reshape(1, 32), rW2, rb2.reshape(1, 16), rW3,
      rb3.reshape(1, 3))

    flags = flags2.reshape(NBLK).astype(jnp.int32)
    mask = mask2.reshape(N)

    wc1, bc1, wc2, bc2 = pl.pallas_call(
        _fuse_body,
        out_shape=(jax.ShapeDtypeStruct((D, D), jnp.float32),
                   jax.ShapeDtypeStruct((1, D), jnp.float32),
                   jax.ShapeDtypeStruct((D, D), jnp.float32),
                   jax.ShapeDtypeStruct((1, D), jnp.float32)),
        interpret=_INTERPRET,
    )(W0, b0.reshape(1, D), W1, b1.reshape(1, D),
      W2, b2.reshape(1, D), W3, b3.reshape(1, D))

    grid_spec = pltpu.PrefetchScalarGridSpec(
        num_scalar_prefetch=1,
        grid=(NBLK,),
        in_specs=[
            pl.BlockSpec((BLK, D), lambda i, flags: (i, 0)),
            pl.BlockSpec((BLK, 1), lambda i, flags: (i, 0)),
            _full((D, D)), _full((1, D)),
            _full((D, D)), _full((1, D)),
            _full((D, D)), _full((1, D)),
        ],
        out_specs=pl.BlockSpec((BLK, D), lambda i, flags: (i, 0)),
    )
    out = pl.pallas_call(
        _chain_body,
        grid_spec=grid_spec,
        out_shape=jax.ShapeDtypeStruct((N, D), jnp.float32),
        interpret=_INTERPRET,
    )(flags, x, mask.reshape(N, 1),
      W0, b0.reshape(1, D), wc1, bc1, wc2, bc2)
    return out, mask


# speedup vs baseline: 1.7502x; 1.0607x over previous
import jax
import jax.numpy as jnp
from jax.experimental import pallas as pl
from jax.experimental.pallas import tpu as pltpu

N = 32768
D = 768
BLK = 2048
NBLK = N // BLK


def _router_body(u_ref, w1_ref, b1_ref, w2_ref, b2_ref, w3_ref, b3_ref,
                 mask_ref, flags_ref):
    u = u_ref[...]
    umin = jnp.min(u)
    umax = jnp.max(u)
    un = (u - umin) / (umax - umin + 1e-8)
    acc = [b2_ref[0, k] * jnp.ones_like(un) for k in range(16)]
    for j in range(32):
        hj = jax.nn.relu(un * w1_ref[0, j] + b1_ref[0, j])
        for k in range(16):
            acc[k] = acc[k] + hj * w2_ref[j, k]
    l0 = jnp.full_like(un, b3_ref[0, 0])
    l1 = jnp.full_like(un, b3_ref[0, 1])
    l2 = jnp.full_like(un, b3_ref[0, 2])
    for k in range(16):
        hk = jax.nn.relu(acc[k])
        l0 = l0 + hk * w3_ref[k, 0]
        l1 = l1 + hk * w3_ref[k, 1]
        l2 = l2 + hk * w3_ref[k, 2]
    d = jnp.where((l1 > l0) & (l1 >= l2), 1.0,
                  jnp.where((l2 > l0) & (l2 > l1), 2.0, 0.0))
    mask_ref[...] = d
    any0 = jnp.max(jnp.where(d == 0.0, 1.0, 0.0), axis=1, keepdims=True)
    any1 = jnp.max(jnp.where(d == 1.0, 1.0, 0.0), axis=1, keepdims=True)
    any2 = jnp.max(jnp.where(d == 2.0, 1.0, 0.0), axis=1, keepdims=True)
    flags_ref[...] = any0 + 2.0 * any1 + 4.0 * any2


def _fuse_body(w0_ref, b0_ref, w1_ref, b1_ref, w2_ref, b2_ref, w3_ref, b3_ref,
               wc1_ref, bc1_ref, wc2_ref, bc2_ref):
    w01 = jnp.dot(w0_ref[...], w1_ref[...], preferred_element_type=jnp.float32)
    wc1_ref[...] = w01
    bc1 = jnp.dot(b0_ref[...], w1_ref[...], preferred_element_type=jnp.float32) + b1_ref[...]
    bc1_ref[...] = bc1
    w012 = jnp.dot(w01, w2_ref[...], preferred_element_type=jnp.float32)
    wc2_ref[...] = jnp.dot(w012, w3_ref[...], preferred_element_type=jnp.float32)
    bc2 = jnp.dot(bc1, w2_ref[...], preferred_element_type=jnp.float32) + b2_ref[...]
    bc2_ref[...] = jnp.dot(bc2, w3_ref[...], preferred_element_type=jnp.float32) + b3_ref[...]


def _chain_body(flags_ref, x_ref, m_ref,
                w0_ref, b0_ref, wc1_ref, bc1_ref, wc2_ref, bc2_ref,
                out_ref):
    i = pl.program_id(0)
    f = flags_ref[i]
    x = x_ref[...]
    m = m_ref[...]

    @pl.when((f & 1) != 0)
    def _():
        out_ref[...] = jnp.dot(x, w0_ref[...], preferred_element_type=jnp.float32) + b0_ref[...]

    @pl.when((f & 2) != 0)
    def _():
        s1 = jnp.dot(x, wc1_ref[...], preferred_element_type=jnp.float32) + bc1_ref[...]

        @pl.when((f & 1) != 0)
        def _():
            out_ref[...] = jnp.where(m == 1.0, s1, out_ref[...])

        @pl.when((f & 1) == 0)
        def _():
            out_ref[...] = s1

    @pl.when((f & 4) != 0)
    def _():
        s2 = jnp.dot(x, wc2_ref[...], preferred_element_type=jnp.float32) + bc2_ref[...]

        @pl.when((f & 3) != 0)
        def _():
            out_ref[...] = jnp.where(m == 2.0, s2, out_ref[...])

        @pl.when((f & 3) == 0)
        def _():
            out_ref[...] = s2


_INTERPRET = False


def _full(shape):
    return pl.BlockSpec(shape, lambda i, flags: (0, 0))


def kernel(x, current_uncertainty, rW1, rb1, rW2, rb2, rW3, rb3,
           W0, b0, W1, b1, W2, b2, W3, b3):
    u2 = current_uncertainty.reshape(NBLK, BLK)
    mask2, flags2 = pl.pallas_call(
        _router_body,
        out_shape=(jax.ShapeDtypeStruct((NBLK, BLK), jnp.float32),
                   jax.ShapeDtypeStruct((NBLK, 1), jnp.float32)),
        interpret=_INTERPRET,
    )(u2, rW1, rb1.reshape(1, 32), rW2, rb2.reshape(1, 16), rW3,
      rb3.reshape(1, 3))

    flags = flags2.reshape(NBLK).astype(jnp.int32)
    mask = mask2.reshape(N)

    wc1, bc1, wc2, bc2 = pl.pallas_call(
        _fuse_body,
        out_shape=(jax.ShapeDtypeStruct((D, D), jnp.float32),
                   jax.ShapeDtypeStruct((1, D), jnp.float32),
                   jax.ShapeDtypeStruct((D, D), jnp.float32),
                   jax.ShapeDtypeStruct((1, D), jnp.float32)),
        interpret=_INTERPRET,
    )(W0, b0.reshape(1, D), W1, b1.reshape(1, D),
      W2, b2.reshape(1, D), W3, b3.reshape(1, D))

    grid_spec = pltpu.PrefetchScalarGridSpec(
        num_scalar_prefetch=1,
        grid=(NBLK,),
        in_specs=[
            pl.BlockSpec((BLK, D), lambda i, flags: (i, 0)),
            pl.BlockSpec((BLK, 1), lambda i, flags: (i, 0)),
            _full((D, D)), _full((1, D)),
            _full((D, D)), _full((1, D)),
            _full((D, D)), _full((1, D)),
        ],
        out_specs=pl.BlockSpec((BLK, D), lambda i, flags: (i, 0)),
    )
    out = pl.pallas_call(
        _chain_body,
        grid_spec=grid_spec,
        out_shape=jax.ShapeDtypeStruct((N, D), jnp.float32),
        interpret=_INTERPRET,
    )(flags, x, mask.reshape(N, 1),
      W0, b0.reshape(1, D), wc1, bc1, wc2, bc2)
    return out, mask
